# baseline (device time: 117061 ns/iter reference)
import jax
import jax.numpy as jnp
from jax import lax
from jax.experimental import pallas as pl
from jax.experimental.pallas import tpu as pltpu

N_DEV = 4


def kernel(x, w_mat):
    w_mat = w_mat.astype(jnp.bfloat16)
    m_per, k = x.shape
    k2, n_per = w_mat.shape
    nh = n_per // 2

    def body(x_hbm, w_ref, out_hbm, xf32, xbf,
             wL, wR, wOa, wOb,
             bBl, bBr, bBoa, bBob,
             fwdA, fwdB,
             bInL, bInR, bOa, bOb,
             st_me, st_l, st_r, st_oa, st_ob,
             send_sems, recv_sems, local_sems):
        my_pos = lax.axis_index("i")
        left = (my_pos - 1) % N_DEV
        right = (my_pos + 1) % N_DEV
        opp = (my_pos + 2) % N_DEV

        def rdma(src, dst, i, dev):
            return pltpu.make_async_remote_copy(
                src_ref=src, dst_ref=dst,
                send_sem=send_sems.at[i], recv_sem=recv_sems.at[i],
                device_id=(dev,), device_id_type=pl.DeviceIdType.MESH,
            )

        a = pl.ds(0, nh)
        b = pl.ds(nh, nh)

        x_load = pltpu.make_async_copy(x_hbm, xf32, local_sems.at[0])
        x_load.start()

        barrier_sem = pltpu.get_barrier_semaphore()
        for nbr in [left, right]:
            pl.semaphore_signal(
                barrier_sem, inc=1,
                device_id=(nbr,), device_id_type=pl.DeviceIdType.MESH,
            )
        pl.semaphore_wait(barrier_sem, 2)

        sW_ra = rdma(w_ref.at[:, a], wL.at[:, a], 0, right)
        sW_lb = rdma(w_ref.at[:, b], wR.at[:, b], 2, left)
        sW_ra.start()
        sW_lb.start()

        x_load.wait()
        xbf[:, :] = xf32[:, :].astype(jnp.bfloat16)
        st_me[:, :] = jnp.dot(
            xbf[:, :], w_ref[:, :], preferred_element_type=jnp.float32
        )
        c_me = pltpu.make_async_copy(
            st_me, out_hbm.at[pl.ds(my_pos * m_per, m_per), :],
            local_sems.at[1],
        )
        c_me.start()

        sW_ra.wait_recv()
        fW_r = rdma(wL.at[:, a], wOa, 4, right)
        fW_r.start()
        sW_rb = rdma(w_ref.at[:, b], wL.at[:, b], 1, right)
        sW_rb.start()

        sW_lb.wait_recv()
        fW_l = rdma(wR.at[:, b], wOb, 5, left)
        fW_l.start()
        sW_la = rdma(w_ref.at[:, a], wR.at[:, a], 3, left)
        sW_la.start()

        bBl[:, a] = jnp.dot(
            xbf[:, :], wL[:, a], preferred_element_type=jnp.float32
        ).astype(jnp.bfloat16)
        sBl_a = rdma(bBl.at[:, a], bInR.at[:, a], 6, left)
        sBl_a.start()
        bBr[:, b] = jnp.dot(
            xbf[:, :], wR[:, b], preferred_element_type=jnp.float32
        ).astype(jnp.bfloat16)
        sBr_b = rdma(bBr.at[:, b], bInL.at[:, b], 8, right)
        sBr_b.start()

        fW_r.wait_recv()
        bBoa[:, :] = jnp.dot(
            xbf[:, :], wOa[:, :], preferred_element_type=jnp.float32
        ).astype(jnp.bfloat16)
        sBo_a = rdma(bBoa, fwdA, 10, right)
        sBo_a.start()
        fW_l.wait_recv()
        bBob[:, :] = jnp.dot(
            xbf[:, :], wOb[:, :], preferred_element_type=jnp.float32
        ).astype(jnp.bfloat16)
        sBo_b = rdma(bBob, fwdB, 11, left)
        sBo_b.start()

        sBl_a.wait_recv()
        st_r[:, a] = bInR[:, a].astype(jnp.float32)
        c_ra = pltpu.make_async_copy(
            st_r.at[:, a], out_hbm.at[pl.ds(right * m_per, m_per), a],
            local_sems.at[2],
        )
        c_ra.start()
        sBr_b.wait_recv()
        st_l[:, b] = bInL[:, b].astype(jnp.float32)
        c_lb = pltpu.make_async_copy(
            st_l.at[:, b], out_hbm.at[pl.ds(left * m_per, m_per), b],
            local_sems.at[3],
        )
        c_lb.start()

        sBo_a.wait_recv()
        fB_r = rdma(fwdA, bOa, 12, right)
        fB_r.start()
        sBo_b.wait_recv()
        fB_l = rdma(fwdB, bOb, 13, left)
        fB_l.start()

        sW_rb.wait_recv()
        bBl[:, b] = jnp.dot(
            xbf[:, :], wL[:, b], preferred_element_type=jnp.float32
        ).astype(jnp.bfloat16)
        sBl_b = rdma(bBl.at[:, b], bInR.at[:, b], 7, left)
        sBl_b.start()
        sW_la.wait_recv()
        bBr[:, a] = jnp.dot(
            xbf[:, :], wR[:, a], preferred_element_type=jnp.float32
        ).astype(jnp.bfloat16)
        sBr_a = rdma(bBr.at[:, a], bInL.at[:, a], 9, right)
        sBr_a.start()

        fB_r.wait_recv()
        st_oa[:, :] = bOa[:, :].astype(jnp.float32)
        c_oa = pltpu.make_async_copy(
            st_oa, out_hbm.at[pl.ds(opp * m_per, m_per), a],
            local_sems.at[4],
        )
        c_oa.start()
        fB_l.wait_recv()
        st_ob[:, :] = bOb[:, :].astype(jnp.float32)
        c_ob = pltpu.make_async_copy(
            st_ob, out_hbm.at[pl.ds(opp * m_per, m_per), b],
            local_sems.at[5],
        )
        c_ob.start()

        sBl_b.wait_recv()
        st_r[:, b] = bInR[:, b].astype(jnp.float32)
        c_rb = pltpu.make_async_copy(
            st_r.at[:, b], out_hbm.at[pl.ds(right * m_per, m_per), b],
            local_sems.at[6],
        )
        c_rb.start()
        sBr_a.wait_recv()
        st_l[:, a] = bInL[:, a].astype(jnp.float32)
        c_la = pltpu.make_async_copy(
            st_l.at[:, a], out_hbm.at[pl.ds(left * m_per, m_per), a],
            local_sems.at[7],
        )
        c_la.start()

        for c in [c_me, c_ra, c_lb, c_oa, c_ob, c_rb, c_la]:
            c.wait()
        for s in [sW_ra, sW_rb, sW_la, sW_lb, fW_r, fW_l,
                  sBl_a, sBl_b, sBr_a, sBr_b,
                  sBo_a, sBo_b, fB_r, fB_l]:
            s.wait_send()

    bf = jnp.bfloat16
    f32 = jnp.float32
    return pl.pallas_call(
        body,
        out_shape=jax.ShapeDtypeStruct((N_DEV * m_per, n_per), f32),
        in_specs=[
            pl.BlockSpec(memory_space=pl.ANY),
            pl.BlockSpec(memory_space=pltpu.VMEM),
        ],
        out_specs=pl.BlockSpec(memory_space=pl.ANY),
        scratch_shapes=[
            pltpu.VMEM((m_per, k), f32),
            pltpu.VMEM((m_per, k), bf),
            pltpu.VMEM((k, n_per), bf),
            pltpu.VMEM((k, n_per), bf),
            pltpu.VMEM((k, nh), bf),
            pltpu.VMEM((k, nh), bf),
            pltpu.VMEM((m_per, n_per), bf),
            pltpu.VMEM((m_per, n_per), bf),
            pltpu.VMEM((m_per, nh), bf),
            pltpu.VMEM((m_per, nh), bf),
            pltpu.VMEM((m_per, nh), bf),
            pltpu.VMEM((m_per, nh), bf),
            pltpu.VMEM((m_per, n_per), bf),
            pltpu.VMEM((m_per, n_per), bf),
            pltpu.VMEM((m_per, nh), bf),
            pltpu.VMEM((m_per, nh), bf),
            pltpu.VMEM((m_per, n_per), f32),
            pltpu.VMEM((m_per, n_per), f32),
            pltpu.VMEM((m_per, n_per), f32),
            pltpu.VMEM((m_per, nh), f32),
            pltpu.VMEM((m_per, nh), f32),
            pltpu.SemaphoreType.DMA((14,)),
            pltpu.SemaphoreType.DMA((14,)),
            pltpu.SemaphoreType.DMA((8,)),
        ],
        compiler_params=pltpu.CompilerParams(
            collective_id=0, vmem_limit_bytes=100 * 1024 * 1024,
        ),
    )(x, w_mat)


# device time: 113493 ns/iter; 1.0314x vs baseline; 1.0314x over previous
import jax
import jax.numpy as jnp
from jax import lax
from jax.experimental import pallas as pl
from jax.experimental.pallas import tpu as pltpu

N_DEV = 4


def kernel(x, w_mat):
    w_mat = w_mat.astype(jnp.bfloat16)
    m_per, k = x.shape
    k2, n_per = w_mat.shape
    n_half = n_per // 2

    def body(x_hbm, w_ref, out_hbm, xf32, xbf,
             wL, wR, wOa, wOb,
             bBl, bBr, bBoa, bBob,
             fwdA, fwdB,
             bInL, bInR, bOa, bOb,
             st_me, st_l, st_r, st_oa, st_ob,
             send_sems, recv_sems, local_sems):
        my_pos = lax.axis_index("i")
        left = (my_pos - 1) % N_DEV
        right = (my_pos + 1) % N_DEV
        opp = (my_pos + 2) % N_DEV

        def rdma(src, dst, i, dev):
            return pltpu.make_async_remote_copy(
                src_ref=src, dst_ref=dst,
                send_sem=send_sems.at[i], recv_sem=recv_sems.at[i],
                device_id=(dev,), device_id_type=pl.DeviceIdType.MESH,
            )

        x_load = pltpu.make_async_copy(x_hbm, xf32, local_sems.at[0])
        x_load.start()

        barrier_sem = pltpu.get_barrier_semaphore()
        for nbr in [left, right]:
            pl.semaphore_signal(
                barrier_sem, inc=1,
                device_id=(nbr,), device_id_type=pl.DeviceIdType.MESH,
            )
        pl.semaphore_wait(barrier_sem, 2)

        sW_r = rdma(w_ref, wL, 0, right)
        sW_l = rdma(w_ref, wR, 1, left)
        sW_r.start()
        sW_l.start()

        x_load.wait()
        xbf[:, :] = xf32[:, :].astype(jnp.bfloat16)
        st_me[:, :] = jnp.dot(
            xbf[:, :], w_ref[:, :], preferred_element_type=jnp.float32
        )
        c_me = pltpu.make_async_copy(
            st_me, out_hbm.at[pl.ds(my_pos * m_per, m_per), :],
            local_sems.at[1],
        )
        c_me.start()

        sW_r.wait_recv()
        fW_r = rdma(wL.at[:, pl.ds(0, n_half)], wOa, 2, right)
        fW_r.start()
        sW_l.wait_recv()
        fW_l = rdma(wR.at[:, pl.ds(n_half, n_half)], wOb, 3, left)
        fW_l.start()

        bBl[:, :] = jnp.dot(
            xbf[:, :], wL[:, :], preferred_element_type=jnp.float32
        ).astype(jnp.bfloat16)
        sB_l = rdma(bBl, bInR, 4, left)
        sB_l.start()
        bBr[:, :] = jnp.dot(
            xbf[:, :], wR[:, :], preferred_element_type=jnp.float32
        ).astype(jnp.bfloat16)
        sB_r = rdma(bBr, bInL, 5, right)
        sB_r.start()

        fW_r.wait_recv()
        bBoa[:, :] = jnp.dot(
            xbf[:, :], wOa[:, :], preferred_element_type=jnp.float32
        ).astype(jnp.bfloat16)
        sBo_a = rdma(bBoa, fwdA, 6, right)
        sBo_a.start()
        fW_l.wait_recv()
        bBob[:, :] = jnp.dot(
            xbf[:, :], wOb[:, :], preferred_element_type=jnp.float32
        ).astype(jnp.bfloat16)
        sBo_b = rdma(bBob, fwdB, 7, left)
        sBo_b.start()

        sB_l.wait_recv()
        st_r[:, :] = bInR[:, :].astype(jnp.float32)
        c_r = pltpu.make_async_copy(
            st_r, out_hbm.at[pl.ds(right * m_per, m_per), :],
            local_sems.at[2],
        )
        c_r.start()
        sB_r.wait_recv()
        st_l[:, :] = bInL[:, :].astype(jnp.float32)
        c_l = pltpu.make_async_copy(
            st_l, out_hbm.at[pl.ds(left * m_per, m_per), :],
            local_sems.at[3],
        )
        c_l.start()

        sBo_a.wait_recv()
        fB_r = rdma(fwdA, bOa, 8, right)
        fB_r.start()
        sBo_b.wait_recv()
        fB_l = rdma(fwdB, bOb, 9, left)
        fB_l.start()

        fB_r.wait_recv()
        st_oa[:, :] = bOa[:, :].astype(jnp.float32)
        c_oa = pltpu.make_async_copy(
            st_oa, out_hbm.at[pl.ds(opp * m_per, m_per), pl.ds(0, n_half)],
            local_sems.at[4],
        )
        c_oa.start()
        fB_l.wait_recv()
        st_ob[:, :] = bOb[:, :].astype(jnp.float32)
        c_ob = pltpu.make_async_copy(
            st_ob,
            out_hbm.at[pl.ds(opp * m_per, m_per), pl.ds(n_half, n_half)],
            local_sems.at[5],
        )
        c_ob.start()

        for c in [c_me, c_r, c_l, c_oa, c_ob]:
            c.wait()
        for s in [sW_r, sW_l, fW_r, fW_l, sB_l, sB_r,
                  sBo_a, sBo_b, fB_r, fB_l]:
            s.wait_send()

    bf = jnp.bfloat16
    f32 = jnp.float32
    return pl.pallas_call(
        body,
        out_shape=jax.ShapeDtypeStruct((N_DEV * m_per, n_per), f32),
        in_specs=[
            pl.BlockSpec(memory_space=pl.ANY),
            pl.BlockSpec(memory_space=pltpu.VMEM),
        ],
        out_specs=pl.BlockSpec(memory_space=pl.ANY),
        scratch_shapes=[
            pltpu.VMEM((m_per, k), f32),
            pltpu.VMEM((m_per, k), bf),
            pltpu.VMEM((k, n_per), bf),
            pltpu.VMEM((k, n_per), bf),
            pltpu.VMEM((k, n_half), bf),
            pltpu.VMEM((k, n_half), bf),
            pltpu.VMEM((m_per, n_per), bf),
            pltpu.VMEM((m_per, n_per), bf),
            pltpu.VMEM((m_per, n_half), bf),
            pltpu.VMEM((m_per, n_half), bf),
            pltpu.VMEM((m_per, n_half), bf),
            pltpu.VMEM((m_per, n_half), bf),
            pltpu.VMEM((m_per, n_per), bf),
            pltpu.VMEM((m_per, n_per), bf),
            pltpu.VMEM((m_per, n_half), bf),
            pltpu.VMEM((m_per, n_half), bf),
            pltpu.VMEM((m_per, n_per), f32),
            pltpu.VMEM((m_per, n_per), f32),
            pltpu.VMEM((m_per, n_per), f32),
            pltpu.VMEM((m_per, n_half), f32),
            pltpu.VMEM((m_per, n_half), f32),
            pltpu.SemaphoreType.DMA((10,)),
            pltpu.SemaphoreType.DMA((10,)),
            pltpu.SemaphoreType.DMA((6,)),
        ],
        compiler_params=pltpu.CompilerParams(
            collective_id=0, vmem_limit_bytes=100 * 1024 * 1024,
        ),
    )(x, w_mat)
